# Initial kernel scaffold; baseline (speedup 1.0000x reference)
import jax
import jax.numpy as jnp
from jax.experimental import pallas as pl


def kernel(inp):
    raise NotImplementedError("write your pallas kernel here")


# speedup vs baseline: 4.9652x; 4.9652x over previous
import functools

import jax
import jax.numpy as jnp
from jax import lax
from jax.experimental import pallas as pl
from jax.experimental.pallas import tpu as pltpu
from jax.experimental.pallas import tpu_sc as plsc

_L = 16


def _sort4(a, b, c, d):
    lo0, hi0 = jnp.minimum(a, c), jnp.maximum(a, c)
    lo1, hi1 = jnp.minimum(b, d), jnp.maximum(b, d)
    r0 = jnp.minimum(lo0, lo1)
    r3 = jnp.maximum(hi0, hi1)
    m0 = jnp.maximum(lo0, lo1)
    m1 = jnp.minimum(hi0, hi1)
    r1 = jnp.minimum(m0, m1)
    r2 = jnp.maximum(m0, m1)
    return r0, r1, r2, r3


def kernel(inp):
    b, c, h, w = inp.shape
    h2, w2 = h // 2, w // 2
    n = b * c
    img = h * w
    oimg = 4 * h2 * w2
    ngrp = w2 // _L

    info = plsc.get_sparse_core_info()
    nw = info.num_cores * info.num_subcores
    ch_per_w = n // nw
    assert n % nw == 0 and w2 % _L == 0

    x = inp.reshape(n, img)

    mesh = plsc.VectorSubcoreMesh(core_axis_name="core", subcore_axis_name="sub")

    @functools.partial(
        pl.kernel,
        out_type=jax.ShapeDtypeStruct((n, oimg), jnp.float32),
        mesh=mesh,
        scratch_types=[
            pltpu.VMEM((img,), jnp.float32),
            pltpu.VMEM((oimg,), jnp.float32),
        ],
        compiler_params=pltpu.CompilerParams(needs_layout_passes=False),
    )
    def run(x_hbm, out_hbm, in_v, out_v):
        wid = lax.axis_index("sub") * info.num_cores + lax.axis_index("core")
        lane2 = lax.iota(jnp.int32, _L) * 2

        def do_channel(ci, carry):
            ch = wid * ch_per_w + ci
            pltpu.sync_copy(x_hbm.at[ch], in_v)

            def do_rowpair(p, carry2):
                base = p * (2 * w)
                obase = p * w2
                for g in range(ngrp):
                    ia = jnp.broadcast_to(base + 2 * _L * g, (_L,)) + lane2
                    va = plsc.load_gather(in_v, [ia])
                    vb = plsc.load_gather(in_v, [ia + 1])
                    vc = plsc.load_gather(in_v, [ia + w])
                    vd = plsc.load_gather(in_v, [ia + (w + 1)])
                    s0, s1, s2, s3 = _sort4(va, vb, vc, vd)
                    o = obase + g * _L
                    out_v[pl.ds(o, _L)] = s0
                    out_v[pl.ds(o + h2 * w2, _L)] = s1
                    out_v[pl.ds(o + 2 * h2 * w2, _L)] = s2
                    out_v[pl.ds(o + 3 * h2 * w2, _L)] = s3
                return carry2

            lax.fori_loop(0, h2, do_rowpair, 0)
            pltpu.sync_copy(out_v, out_hbm.at[ch])
            return carry

        lax.fori_loop(0, ch_per_w, do_channel, 0)

    out = run(x)
    return out.reshape(b, 4 * c, h2, w2)


# speedup vs baseline: 5.3082x; 1.0691x over previous
import functools

import jax
import jax.numpy as jnp
from jax import lax
from jax.experimental import pallas as pl
from jax.experimental.pallas import tpu as pltpu
from jax.experimental.pallas import tpu_sc as plsc

_L = 16


def _sort4(a, b, c, d):
    lo0, hi0 = jnp.minimum(a, c), jnp.maximum(a, c)
    lo1, hi1 = jnp.minimum(b, d), jnp.maximum(b, d)
    r0 = jnp.minimum(lo0, lo1)
    r3 = jnp.maximum(hi0, hi1)
    m0 = jnp.maximum(lo0, lo1)
    m1 = jnp.minimum(hi0, hi1)
    r1 = jnp.minimum(m0, m1)
    r2 = jnp.maximum(m0, m1)
    return r0, r1, r2, r3


def kernel(inp):
    b, c, h, w = inp.shape
    h2, w2 = h // 2, w // 2
    n = b * c
    img = h * w
    qimg = h2 * w2
    ngrp = w2 // _L

    info = plsc.get_sparse_core_info()
    nw = info.num_cores * info.num_subcores
    ch_per_w = n // nw
    assert n % nw == 0 and w2 % _L == 0 and h2 % 2 == 0

    hp = h2 // 2
    blk_in = hp * 2 * w
    blk_out = hp * w2
    nblk = 2 * ch_per_w

    x = inp.reshape(n, img)

    mesh = plsc.VectorSubcoreMesh(core_axis_name="core", subcore_axis_name="sub")

    @functools.partial(
        pl.kernel,
        out_type=jax.ShapeDtypeStruct((n, 4, qimg), jnp.float32),
        mesh=mesh,
        scratch_types=[
            pltpu.VMEM((blk_in,), jnp.float32),
            pltpu.VMEM((blk_in,), jnp.float32),
            pltpu.VMEM((4, blk_out), jnp.float32),
            pltpu.VMEM((4, blk_out), jnp.float32),
            pltpu.SemaphoreType.DMA,
            pltpu.SemaphoreType.DMA,
            pltpu.SemaphoreType.DMA,
            pltpu.SemaphoreType.DMA,
        ],
        compiler_params=pltpu.CompilerParams(needs_layout_passes=False),
    )
    def run(x_hbm, out_hbm, in_v0, in_v1, out_v0, out_v1, si0, si1, so0, so1):
        in_v = (in_v0, in_v1)
        out_v = (out_v0, out_v1)
        wid = lax.axis_index("sub") * info.num_cores + lax.axis_index("core")
        ch0 = wid * ch_per_w
        lane2 = lax.iota(jnp.int32, _L) * 2
        col0 = [lane2 + 2 * _L * g for g in range(ngrp)]
        sin = (si0, si1)
        sout = (so0, so1)

        def in_copy(t, buf, sem):
            ch = ch0 + t // 2
            off = (t % 2) * blk_in
            return pltpu.make_async_copy(
                x_hbm.at[ch, pl.ds(off, blk_in)], in_v[buf], sem)

        def out_copy(t, buf, sem):
            ch = ch0 + t // 2
            off = (t % 2) * blk_out
            return pltpu.make_async_copy(
                out_v[buf], out_hbm.at[ch, :, pl.ds(off, blk_out)], sem)

        def compute(buf):
            def do_rowpair(p, carry2):
                pb = jnp.broadcast_to(p * (2 * w), (_L,))
                obase = p * w2
                for g in range(ngrp):
                    ia = pb + col0[g]
                    va = plsc.load_gather(in_v[buf], [ia])
                    vb = plsc.load_gather(in_v[buf], [ia + 1])
                    vc = plsc.load_gather(in_v[buf], [ia + w])
                    vd = plsc.load_gather(in_v[buf], [ia + (w + 1)])
                    s0, s1, s2, s3 = _sort4(va, vb, vc, vd)
                    o = obase + g * _L
                    out_v[buf][0, pl.ds(o, _L)] = s0
                    out_v[buf][1, pl.ds(o, _L)] = s1
                    out_v[buf][2, pl.ds(o, _L)] = s2
                    out_v[buf][3, pl.ds(o, _L)] = s3
                return carry2

            lax.fori_loop(0, hp, do_rowpair, 0)

        in_copy(0, 0, sin[0]).start()

        def body(i, carry):
            t0 = i * 2
            for buf in range(2):
                t = t0 + buf
                nxt = 1 - buf

                @pl.when(t + 1 < nblk)
                def _():
                    in_copy(t + 1, nxt, sin[nxt]).start()

                in_copy(t, buf, sin[buf]).wait()

                @pl.when(t >= 2)
                def _():
                    out_copy(t - 2, buf, sout[buf]).wait()

                compute(buf)
                out_copy(t, buf, sout[buf]).start()
            return carry

        lax.fori_loop(0, nblk // 2, body, 0)
        out_copy(nblk - 2, 0, sout[0]).wait()
        out_copy(nblk - 1, 1, sout[1]).wait()

    out = run(x)
    return out.reshape(b, 4 * c, h2, w2)


# speedup vs baseline: 5.7495x; 1.0831x over previous
import functools

import jax
import jax.numpy as jnp
from jax import lax
from jax.experimental import pallas as pl
from jax.experimental.pallas import tpu as pltpu
from jax.experimental.pallas import tpu_sc as plsc

_L = 16


def _sort4(a, b, c, d):
    lo0, hi0 = jnp.minimum(a, c), jnp.maximum(a, c)
    lo1, hi1 = jnp.minimum(b, d), jnp.maximum(b, d)
    r0 = jnp.minimum(lo0, lo1)
    r3 = jnp.maximum(hi0, hi1)
    m0 = jnp.maximum(lo0, lo1)
    m1 = jnp.minimum(hi0, hi1)
    r1 = jnp.minimum(m0, m1)
    r2 = jnp.maximum(m0, m1)
    return r0, r1, r2, r3


def kernel(inp):
    b, c, h, w = inp.shape
    h2, w2 = h // 2, w // 2
    n = b * c
    img = h * w
    qimg = h2 * w2
    ngrp = w2 // _L

    info = plsc.get_sparse_core_info()
    nw = info.num_cores * info.num_subcores
    ch_per_w = n // nw
    assert n % nw == 0 and w2 % _L == 0 and h2 % 2 == 0

    hp = h2 // 2
    blk_in = hp * 2 * w
    blk_out = hp * w2
    nblk = 2 * ch_per_w

    x = inp.reshape(n, img)

    mesh = plsc.VectorSubcoreMesh(core_axis_name="core", subcore_axis_name="sub")

    @functools.partial(
        pl.kernel,
        out_type=jax.ShapeDtypeStruct((n, 4, qimg), jnp.float32),
        mesh=mesh,
        scratch_types=[
            pltpu.VMEM((blk_in,), jnp.float32),
            pltpu.VMEM((blk_in,), jnp.float32),
            pltpu.VMEM((4, blk_out), jnp.float32),
            pltpu.VMEM((4, blk_out), jnp.float32),
            pltpu.SemaphoreType.DMA,
            pltpu.SemaphoreType.DMA,
            pltpu.SemaphoreType.DMA,
            pltpu.SemaphoreType.DMA,
        ],
        compiler_params=pltpu.CompilerParams(needs_layout_passes=False),
    )
    def run(x_hbm, out_hbm, in_v0, in_v1, out_v0, out_v1, si0, si1, so0, so1):
        in_v = (in_v0, in_v1)
        out_v = (out_v0, out_v1)
        wid = lax.axis_index("sub") * info.num_cores + lax.axis_index("core")
        ch0 = wid * ch_per_w
        lane2 = lax.iota(jnp.int32, _L) * 2
        col0 = [lane2 + 2 * _L * g for g in range(ngrp)]
        sin = (si0, si1)
        sout = (so0, so1)

        def in_copy(t, buf, sem):
            ch = ch0 + t // 2
            off = (t % 2) * blk_in
            return pltpu.make_async_copy(
                x_hbm.at[ch, pl.ds(off, blk_in)], in_v[buf], sem)

        def out_copy(t, buf, sem):
            ch = ch0 + t // 2
            off = (t % 2) * blk_out
            return pltpu.make_async_copy(
                out_v[buf], out_hbm.at[ch, :, pl.ds(off, blk_out)], sem)

        def compute(buf):
            @plsc.parallel_loop(0, hp, step=1, unroll=4)
            def do_rowpair(p):
                pb = jnp.broadcast_to(p * (2 * w), (_L,))
                obase = p * w2
                for g in range(ngrp):
                    ia = pb + col0[g]
                    va = plsc.load_gather(in_v[buf], [ia])
                    vb = plsc.load_gather(in_v[buf], [ia + 1])
                    vc = plsc.load_gather(in_v[buf], [ia + w])
                    vd = plsc.load_gather(in_v[buf], [ia + (w + 1)])
                    s0, s1, s2, s3 = _sort4(va, vb, vc, vd)
                    o = obase + g * _L
                    out_v[buf][0, pl.ds(o, _L)] = s0
                    out_v[buf][1, pl.ds(o, _L)] = s1
                    out_v[buf][2, pl.ds(o, _L)] = s2
                    out_v[buf][3, pl.ds(o, _L)] = s3

        in_copy(0, 0, sin[0]).start()

        def body(i, carry):
            t0 = i * 2
            for buf in range(2):
                t = t0 + buf
                nxt = 1 - buf

                @pl.when(t + 1 < nblk)
                def _():
                    in_copy(t + 1, nxt, sin[nxt]).start()

                in_copy(t, buf, sin[buf]).wait()

                @pl.when(t >= 2)
                def _():
                    out_copy(t - 2, buf, sout[buf]).wait()

                compute(buf)
                out_copy(t, buf, sout[buf]).start()
            return carry

        lax.fori_loop(0, nblk // 2, body, 0)
        out_copy(nblk - 2, 0, sout[0]).wait()
        out_copy(nblk - 1, 1, sout[1]).wait()

    out = run(x)
    return out.reshape(b, 4 * c, h2, w2)


# speedup vs baseline: 11.2384x; 1.9547x over previous
import functools

import jax
import jax.numpy as jnp
from jax import lax
from jax.experimental import pallas as pl
from jax.experimental.pallas import tpu as pltpu
from jax.experimental.pallas import tpu_sc as plsc

_L = 16


def _sort4(a, b, c, d):
    lo0, hi0 = jnp.minimum(a, c), jnp.maximum(a, c)
    lo1, hi1 = jnp.minimum(b, d), jnp.maximum(b, d)
    r0 = jnp.minimum(lo0, lo1)
    r3 = jnp.maximum(hi0, hi1)
    m0 = jnp.maximum(lo0, lo1)
    m1 = jnp.minimum(hi0, hi1)
    r1 = jnp.minimum(m0, m1)
    r2 = jnp.maximum(m0, m1)
    return r0, r1, r2, r3


def kernel(inp):
    b, c, h, w = inp.shape
    h2, w2 = h // 2, w // 2
    n = b * c
    ngrp = w2 // _L

    info = plsc.get_sparse_core_info()
    nw = info.num_cores * info.num_subcores
    ch_per_w = n // nw
    assert n % nw == 0 and w2 % _L == 0 and h2 % 2 == 0 and c % (nw // b) == 0

    w_per_b = nw // b
    ch_per_wb = c // w_per_b
    hp = h2 // 2
    hrows = hp * 2
    nblk = 2 * ch_per_wb

    mesh = plsc.VectorSubcoreMesh(core_axis_name="core", subcore_axis_name="sub")

    @functools.partial(
        pl.kernel,
        out_type=jax.ShapeDtypeStruct((b, 4 * c, h2, w2), jnp.float32),
        mesh=mesh,
        scratch_types=[
            pltpu.VMEM((hrows, w), jnp.float32),
            pltpu.VMEM((hrows, w), jnp.float32),
            pltpu.VMEM((4, hp, w2), jnp.float32),
            pltpu.VMEM((4, hp, w2), jnp.float32),
            pltpu.SemaphoreType.DMA,
            pltpu.SemaphoreType.DMA,
            pltpu.SemaphoreType.DMA,
            pltpu.SemaphoreType.DMA,
        ],
        compiler_params=pltpu.CompilerParams(needs_layout_passes=False),
    )
    def run(x_hbm, out_hbm, in_v0, in_v1, out_v0, out_v1, si0, si1, so0, so1):
        in_v = (in_v0, in_v1)
        out_v = (out_v0, out_v1)
        wid = lax.axis_index("sub") * info.num_cores + lax.axis_index("core")
        bb = wid // w_per_b
        cc0 = (wid % w_per_b) * ch_per_wb
        lane2 = lax.iota(jnp.int32, _L) * 2
        cole = [lane2 + 2 * _L * g for g in range(ngrp)]
        colo = [lane2 + 2 * _L * g + 1 for g in range(ngrp)]
        sin = (si0, si1)
        sout = (so0, so1)

        def in_copy(t, buf, sem):
            cc = cc0 + t // 2
            r0 = (t % 2) * hrows
            return pltpu.make_async_copy(
                x_hbm.at[bb, cc, pl.ds(r0, hrows), :], in_v[buf], sem)

        def out_copy(t, buf, sem):
            cc = cc0 + t // 2
            p0 = (t % 2) * hp
            return pltpu.make_async_copy(
                out_v[buf],
                out_hbm.at[bb, pl.ds(cc * 4, 4), pl.ds(p0, hp), :], sem)

        def compute(buf):
            @plsc.parallel_loop(0, hp, step=1, unroll=4)
            def do_rowpair(p):
                rt = jnp.broadcast_to(2 * p, (_L,))
                rb = rt + 1
                for g in range(ngrp):
                    va = plsc.load_gather(in_v[buf], [rt, cole[g]])
                    vb = plsc.load_gather(in_v[buf], [rt, colo[g]])
                    vc = plsc.load_gather(in_v[buf], [rb, cole[g]])
                    vd = plsc.load_gather(in_v[buf], [rb, colo[g]])
                    s0, s1, s2, s3 = _sort4(va, vb, vc, vd)
                    o = g * _L
                    out_v[buf][0, p, pl.ds(o, _L)] = s0
                    out_v[buf][1, p, pl.ds(o, _L)] = s1
                    out_v[buf][2, p, pl.ds(o, _L)] = s2
                    out_v[buf][3, p, pl.ds(o, _L)] = s3

        in_copy(0, 0, sin[0]).start()

        def body(i, carry):
            t0 = i * 2
            for buf in range(2):
                t = t0 + buf
                nxt = 1 - buf

                @pl.when(t + 1 < nblk)
                def _():
                    in_copy(t + 1, nxt, sin[nxt]).start()

                in_copy(t, buf, sin[buf]).wait()

                @pl.when(t >= 2)
                def _():
                    out_copy(t - 2, buf, sout[buf]).wait()

                compute(buf)
                out_copy(t, buf, sout[buf]).start()
            return carry

        lax.fori_loop(0, nblk // 2, body, 0)
        out_copy(nblk - 2, 0, sout[0]).wait()
        out_copy(nblk - 1, 1, sout[1]).wait()

    return run(inp)


# speedup vs baseline: 12.3426x; 1.0983x over previous
import functools

import jax
import jax.numpy as jnp
from jax import lax
from jax.experimental import pallas as pl
from jax.experimental.pallas import tpu as pltpu
from jax.experimental.pallas import tpu_sc as plsc

_L = 16


def _sort4(a, b, c, d):
    lo0, hi0 = jnp.minimum(a, c), jnp.maximum(a, c)
    lo1, hi1 = jnp.minimum(b, d), jnp.maximum(b, d)
    r0 = jnp.minimum(lo0, lo1)
    r3 = jnp.maximum(hi0, hi1)
    m0 = jnp.maximum(lo0, lo1)
    m1 = jnp.minimum(hi0, hi1)
    r1 = jnp.minimum(m0, m1)
    r2 = jnp.maximum(m0, m1)
    return r0, r1, r2, r3


def kernel(inp):
    b, c, h, w = inp.shape
    h2, w2 = h // 2, w // 2
    n = b * c
    ngrp = w2 // _L

    info = plsc.get_sparse_core_info()
    nw = info.num_cores * info.num_subcores
    ch_per_w = n // nw
    assert n % nw == 0 and w2 % _L == 0 and h2 % 2 == 0 and c % (nw // b) == 0

    w_per_b = nw // b
    ch_per_wb = c // w_per_b
    hp = h2 // 2
    hrows = hp * 2
    nblk = 2 * ch_per_wb

    mesh = plsc.VectorSubcoreMesh(core_axis_name="core", subcore_axis_name="sub")

    @functools.partial(
        pl.kernel,
        out_type=jax.ShapeDtypeStruct((b, 4 * c, h2, w2), jnp.float32),
        mesh=mesh,
        scratch_types=[
            pltpu.VMEM((hrows, w), jnp.float32),
            pltpu.VMEM((hrows, w), jnp.float32),
            pltpu.VMEM((4, hp, w2), jnp.float32),
            pltpu.VMEM((4, hp, w2), jnp.float32),
            pltpu.SemaphoreType.DMA,
            pltpu.SemaphoreType.DMA,
            pltpu.SemaphoreType.DMA,
            pltpu.SemaphoreType.DMA,
        ],
        compiler_params=pltpu.CompilerParams(needs_layout_passes=False),
    )
    def run(x_hbm, out_hbm, in_v0, in_v1, out_v0, out_v1, si0, si1, so0, so1):
        in_v = (in_v0, in_v1)
        out_v = (out_v0, out_v1)
        wid = lax.axis_index("sub") * info.num_cores + lax.axis_index("core")
        bb = wid // w_per_b
        cc0 = (wid % w_per_b) * ch_per_wb
        lane2 = lax.iota(jnp.int32, _L) * 2
        cole = [lane2 + 2 * _L * g for g in range(ngrp)]
        colo = [lane2 + 2 * _L * g + 1 for g in range(ngrp)]
        sin = (si0, si1)
        sout = (so0, so1)

        def in_copy(t, buf, sem):
            cc = cc0 + t // 2
            r0 = (t % 2) * hrows
            return pltpu.make_async_copy(
                x_hbm.at[bb, cc, pl.ds(r0, hrows), :], in_v[buf], sem)

        def out_copy(t, buf, sem):
            cc = cc0 + t // 2
            p0 = (t % 2) * hp
            return pltpu.make_async_copy(
                out_v[buf],
                out_hbm.at[bb, pl.ds(cc * 4, 4), pl.ds(p0, hp), :], sem)

        def compute(buf):
            @plsc.parallel_loop(0, hp, step=1, unroll=8)
            def do_rowpair(p):
                rt = jnp.broadcast_to(2 * p, (_L,))
                rb = rt + 1
                for g in range(ngrp):
                    va = plsc.load_gather(in_v[buf], [rt, cole[g]])
                    vb = plsc.load_gather(in_v[buf], [rt, colo[g]])
                    vc = plsc.load_gather(in_v[buf], [rb, cole[g]])
                    vd = plsc.load_gather(in_v[buf], [rb, colo[g]])
                    s0, s1, s2, s3 = _sort4(va, vb, vc, vd)
                    o = g * _L
                    out_v[buf][0, p, pl.ds(o, _L)] = s0
                    out_v[buf][1, p, pl.ds(o, _L)] = s1
                    out_v[buf][2, p, pl.ds(o, _L)] = s2
                    out_v[buf][3, p, pl.ds(o, _L)] = s3

        in_copy(0, 0, sin[0]).start()

        def body(i, carry):
            t0 = i * 2
            for buf in range(2):
                t = t0 + buf
                nxt = 1 - buf

                @pl.when(t + 1 < nblk)
                def _():
                    in_copy(t + 1, nxt, sin[nxt]).start()

                in_copy(t, buf, sin[buf]).wait()

                @pl.when(t >= 2)
                def _():
                    out_copy(t - 2, buf, sout[buf]).wait()

                compute(buf)
                out_copy(t, buf, sout[buf]).start()
            return carry

        lax.fori_loop(0, nblk // 2, body, 0)
        out_copy(nblk - 2, 0, sout[0]).wait()
        out_copy(nblk - 1, 1, sout[1]).wait()

    return run(inp)
